# Initial kernel scaffold; baseline (speedup 1.0000x reference)
import math
import functools

import jax
import jax.numpy as jnp
from jax import lax
from jax.experimental import pallas as pl
from jax.experimental.pallas import tpu as pltpu
from jax.experimental.pallas import tpu_sc as plsc

NC = 2
NS = 16
NW = NC * NS

CHUNK = 100


def _pe_table(seq, d_model):
    position = jnp.arange(0, seq, dtype=jnp.float32)[:, None]
    div_term = jnp.exp(
        jnp.arange(0, d_model, 2, dtype=jnp.float32)
        * -(math.log(10000.0) / d_model)
    )
    pe = jnp.zeros((seq, d_model), dtype=jnp.float32)
    pe = pe.at[:, 0::2].set(jnp.sin(position * div_term))
    pe = pe.at[:, 1::2].set(jnp.cos(position * div_term))
    return pe


def _build_sc_call(n_chunks, d_model, seq):
    mesh = plsc.VectorSubcoreMesh(
        core_axis_name="c", subcore_axis_name="s",
        num_cores=NC, num_subcores=NS,
    )
    total = NW * n_chunks * CHUNK

    @functools.partial(
        pl.kernel,
        out_type=jax.ShapeDtypeStruct((total, d_model), jnp.float32),
        mesh=mesh,
        scratch_types=[
            pltpu.VMEM((n_chunks, CHUNK), jnp.int32),
            pltpu.VMEM((seq, d_model), jnp.float32),
            pltpu.VMEM((CHUNK, d_model), jnp.float32),
            pltpu.SemaphoreType.DMA,
        ],
    )
    def sc_call(idx_hbm, table_hbm, pe_hbm, out_hbm, idx_v, pe_v, rows_v, sem):
        wid = lax.axis_index("s") * NC + lax.axis_index("c")
        pltpu.sync_copy(idx_hbm.at[wid], idx_v)
        pltpu.sync_copy(pe_hbm, pe_v)

        @pl.loop(0, n_chunks)
        def _chunks(g):
            pltpu.async_copy(table_hbm.at[idx_v.at[g]], rows_v, sem).wait()
            p0 = CHUNK * lax.rem(g, 2)

            @pl.loop(0, CHUNK)
            def _rows(r):
                pr = p0 + r
                for cb in range(d_model // 16):
                    sl = pl.ds(cb * 16, 16)
                    rows_v[r, sl] = rows_v[r, sl] + pe_v[pr, sl]

            base = (wid * n_chunks + g) * CHUNK
            pltpu.sync_copy(rows_v, out_hbm.at[pl.ds(base, CHUNK)])

    return sc_call


def kernel(input, table):
    batch, seq = input.shape
    vocab, d_model = table.shape
    total = batch * seq
    assert total % (NW * CHUNK) == 0 and seq == 2 * CHUNK
    n_chunks = total // (NW * CHUNK)

    pe = _pe_table(seq, d_model)
    idx = input.reshape(NW, n_chunks, CHUNK).astype(jnp.int32)
    out = _build_sc_call(n_chunks, d_model, seq)(idx, table, pe)
    return out.reshape(batch, seq, d_model)


# speedup vs baseline: 2.1705x; 2.1705x over previous
import math
import functools

import jax
import jax.numpy as jnp
from jax import lax
from jax.experimental import pallas as pl
from jax.experimental.pallas import tpu as pltpu
from jax.experimental.pallas import tpu_sc as plsc

NC = 2
NS = 16
NW = NC * NS

CHUNK = 128


def _pe_table(seq, d_model):
    position = jnp.arange(0, seq, dtype=jnp.float32)[:, None]
    div_term = jnp.exp(
        jnp.arange(0, d_model, 2, dtype=jnp.float32)
        * -(math.log(10000.0) / d_model)
    )
    pe = jnp.zeros((seq, d_model), dtype=jnp.float32)
    pe = pe.at[:, 0::2].set(jnp.sin(position * div_term))
    pe = pe.at[:, 1::2].set(jnp.cos(position * div_term))
    return pe


def _build_sc_call(n_chunks, d_model, seq):
    mesh = plsc.VectorSubcoreMesh(
        core_axis_name="c", subcore_axis_name="s",
        num_cores=NC, num_subcores=NS,
    )
    total = NW * n_chunks * CHUNK

    @functools.partial(
        pl.kernel,
        out_type=jax.ShapeDtypeStruct((total, d_model), jnp.float32),
        mesh=mesh,
        scratch_types=[
            pltpu.VMEM((n_chunks, CHUNK), jnp.int32),
            pltpu.VMEM((2 * seq, d_model), jnp.float32),
            pltpu.VMEM((CHUNK, d_model), jnp.float32),
            pltpu.SemaphoreType.DMA,
        ],
        compiler_params=pltpu.CompilerParams(use_tc_tiling_on_sc=False),
    )
    def sc_call(idx_hbm, table_hbm, pe2_hbm, out_hbm, idx_v, pe_v, rows_v, sem):
        wid = lax.axis_index("s") * NC + lax.axis_index("c")
        pltpu.sync_copy(idx_hbm.at[wid], idx_v)
        pltpu.sync_copy(pe2_hbm, pe_v)

        @pl.loop(0, n_chunks)
        def _chunks(g):
            pltpu.async_copy(table_hbm.at[idx_v.at[g]], rows_v, sem).wait()
            p0 = lax.rem(g * CHUNK, seq)

            @pl.loop(0, CHUNK)
            def _rows(r):
                pr = p0 + r
                for cb in range(d_model // 16):
                    sl = pl.ds(cb * 16, 16)
                    rows_v[r, sl] = rows_v[r, sl] + pe_v[pr, sl]

            base = (wid * n_chunks + g) * CHUNK
            pltpu.sync_copy(rows_v, out_hbm.at[pl.ds(base, CHUNK)])

    return sc_call


def kernel(input, table):
    batch, seq = input.shape
    vocab, d_model = table.shape
    total = batch * seq
    per_worker = total // NW
    assert total % (NW * CHUNK) == 0 and per_worker % seq == 0
    n_chunks = per_worker // CHUNK

    pe = _pe_table(seq, d_model)
    pe2 = jnp.concatenate([pe, pe], axis=0)
    idx = input.reshape(NW, n_chunks, CHUNK).astype(jnp.int32)
    out = _build_sc_call(n_chunks, d_model, seq)(idx, table, pe2)
    return out.reshape(batch, seq, d_model)


# speedup vs baseline: 3.9641x; 1.8264x over previous
import math
import functools

import jax
import jax.numpy as jnp
from jax import lax
from jax.experimental import pallas as pl
from jax.experimental.pallas import tpu as pltpu
from jax.experimental.pallas import tpu_sc as plsc

NC = 2
NS = 16
NW = NC * NS

CHUNK = 128
NBUF = 4


def _pe_table(seq, d_model):
    position = jnp.arange(0, seq, dtype=jnp.float32)[:, None]
    div_term = jnp.exp(
        jnp.arange(0, d_model, 2, dtype=jnp.float32)
        * -(math.log(10000.0) / d_model)
    )
    pe = jnp.zeros((seq, d_model), dtype=jnp.float32)
    pe = pe.at[:, 0::2].set(jnp.sin(position * div_term))
    pe = pe.at[:, 1::2].set(jnp.cos(position * div_term))
    return pe


def _build_sc_call(n_chunks, d_model, seq):
    mesh = plsc.VectorSubcoreMesh(
        core_axis_name="c", subcore_axis_name="s",
        num_cores=NC, num_subcores=NS,
    )
    total = NW * n_chunks * CHUNK
    n_outer = n_chunks // NBUF

    @functools.partial(
        pl.kernel,
        out_type=jax.ShapeDtypeStruct((total, d_model), jnp.float32),
        mesh=mesh,
        scratch_types=[
            pltpu.VMEM((n_chunks, CHUNK), jnp.int32),
            pltpu.VMEM((2 * seq, d_model), jnp.float32),
        ]
        + [pltpu.VMEM((CHUNK, d_model), jnp.float32) for _ in range(NBUF)]
        + [
            pltpu.SemaphoreType.DMA((NBUF,)),
            pltpu.SemaphoreType.DMA((NBUF,)),
        ],
        compiler_params=pltpu.CompilerParams(use_tc_tiling_on_sc=False),
    )
    def sc_call(idx_hbm, table_hbm, pe2_hbm, out_hbm,
                idx_v, pe_v, *bufs_and_sems):
        bufs = bufs_and_sems[:NBUF]
        gsem, osem = bufs_and_sems[NBUF], bufs_and_sems[NBUF + 1]

        wid = lax.axis_index("s") * NC + lax.axis_index("c")
        pltpu.sync_copy(idx_hbm.at[wid], idx_v)
        pltpu.sync_copy(pe2_hbm, pe_v)

        for b in range(NBUF):
            pltpu.async_copy(table_hbm.at[idx_v.at[b]], bufs[b], gsem.at[b])

        @pl.loop(0, n_outer)
        def _outer(t):
            for b in range(NBUF):
                g = t * NBUF + b
                buf = bufs[b]
                pltpu.make_async_copy(
                    table_hbm.at[idx_v.at[g]], buf, gsem.at[b]).wait()

                p0 = lax.rem(g * CHUNK, seq)

                @plsc.parallel_loop(0, CHUNK, unroll=8)
                def _rows(r):
                    pr = p0 + r
                    for cb in range(d_model // 16):
                        sl = pl.ds(cb * 16, 16)
                        buf[r, sl] = buf[r, sl] + pe_v[pr, sl]

                base = (wid * n_chunks + g) * CHUNK
                dst = out_hbm.at[pl.ds(base, CHUNK)]
                pltpu.async_copy(buf, dst, osem.at[b])

                gn = g + NBUF

                @pl.when(gn < n_chunks)
                def _refill():
                    pltpu.make_async_copy(buf, dst, osem.at[b]).wait()
                    pltpu.async_copy(
                        table_hbm.at[idx_v.at[gn]], buf, gsem.at[b])

        for b in range(NBUF):
            g = n_chunks - NBUF + b
            base = (wid * n_chunks + g) * CHUNK
            pltpu.make_async_copy(
                bufs[b], out_hbm.at[pl.ds(base, CHUNK)], osem.at[b]).wait()

    return sc_call


def kernel(input, table):
    batch, seq = input.shape
    vocab, d_model = table.shape
    total = batch * seq
    per_worker = total // NW
    assert total % (NW * CHUNK) == 0 and per_worker % seq == 0
    n_chunks = per_worker // CHUNK
    assert n_chunks % NBUF == 0

    pe = _pe_table(seq, d_model)
    pe2 = jnp.concatenate([pe, pe], axis=0)
    idx = input.reshape(NW, n_chunks, CHUNK).astype(jnp.int32)
    out = _build_sc_call(n_chunks, d_model, seq)(idx, table, pe2)
    return out.reshape(batch, seq, d_model)


# speedup vs baseline: 4.2168x; 1.0637x over previous
import math
import functools

import jax
import jax.numpy as jnp
from jax import lax
from jax.experimental import pallas as pl
from jax.experimental.pallas import tpu as pltpu
from jax.experimental.pallas import tpu_sc as plsc

NC = 2
NS = 16
NW = NC * NS

CHUNK = 128
NG = 4
NO = 2


def _pe_table(seq, d_model):
    position = jnp.arange(0, seq, dtype=jnp.float32)[:, None]
    div_term = jnp.exp(
        jnp.arange(0, d_model, 2, dtype=jnp.float32)
        * -(math.log(10000.0) / d_model)
    )
    pe = jnp.zeros((seq, d_model), dtype=jnp.float32)
    pe = pe.at[:, 0::2].set(jnp.sin(position * div_term))
    pe = pe.at[:, 1::2].set(jnp.cos(position * div_term))
    return pe


def _build_sc_call(n_chunks, d_model, seq):
    mesh = plsc.VectorSubcoreMesh(
        core_axis_name="c", subcore_axis_name="s",
        num_cores=NC, num_subcores=NS,
    )
    total = NW * n_chunks * CHUNK
    n_outer = n_chunks // NG

    @functools.partial(
        pl.kernel,
        out_type=jax.ShapeDtypeStruct((total, d_model), jnp.float32),
        mesh=mesh,
        scratch_types=[
            pltpu.VMEM((n_chunks, CHUNK), jnp.int32),
            pltpu.VMEM((2 * seq, d_model), jnp.float32),
        ]
        + [pltpu.VMEM((CHUNK, d_model), jnp.float32) for _ in range(NG + NO)]
        + [
            pltpu.SemaphoreType.DMA((NG,)),
            pltpu.SemaphoreType.DMA((NO,)),
        ],
        compiler_params=pltpu.CompilerParams(use_tc_tiling_on_sc=False),
    )
    def sc_call(idx_hbm, table_hbm, pe2_hbm, out_hbm,
                idx_v, pe_v, *bufs_and_sems):
        gin = bufs_and_sems[:NG]
        gout = bufs_and_sems[NG:NG + NO]
        gsem, osem = bufs_and_sems[NG + NO], bufs_and_sems[NG + NO + 1]

        wid = lax.axis_index("s") * NC + lax.axis_index("c")
        pltpu.sync_copy(idx_hbm.at[wid], idx_v)
        pltpu.sync_copy(pe2_hbm, pe_v)

        for b in range(NG):
            pltpu.async_copy(table_hbm.at[idx_v.at[b]], gin[b], gsem.at[b])

        def out_slice(g):
            base = (wid * n_chunks + g) * CHUNK
            return out_hbm.at[pl.ds(base, CHUNK)]

        @pl.loop(0, n_outer)
        def _outer(t):
            for b in range(NG):
                bo = b % NO
                g = t * NG + b
                src = gin[b]
                dst = gout[bo]
                pltpu.make_async_copy(
                    table_hbm.at[idx_v.at[g]], src, gsem.at[b]).wait()

                @pl.when(g >= NO)
                def _wait_prev_out():
                    pltpu.make_async_copy(
                        dst, out_slice(g - NO), osem.at[bo]).wait()

                p0 = lax.rem(g * CHUNK, seq)

                @plsc.parallel_loop(0, CHUNK, unroll=8)
                def _rows(r):
                    pr = p0 + r
                    for cb in range(d_model // 16):
                        sl = pl.ds(cb * 16, 16)
                        dst[r, sl] = src[r, sl] + pe_v[pr, sl]

                gn = g + NG

                @pl.when(gn < n_chunks)
                def _refill():
                    pltpu.async_copy(
                        table_hbm.at[idx_v.at[gn]], src, gsem.at[b])

                pltpu.async_copy(dst, out_slice(g), osem.at[bo])

        for b in range(NO):
            g = n_chunks - NO + b
            pltpu.make_async_copy(
                gout[g % NO], out_slice(g), osem.at[g % NO]).wait()

    return sc_call


def kernel(input, table):
    batch, seq = input.shape
    vocab, d_model = table.shape
    total = batch * seq
    per_worker = total // NW
    assert total % (NW * CHUNK) == 0 and per_worker % seq == 0
    n_chunks = per_worker // CHUNK
    assert n_chunks % NG == 0 and NG % NO == 0

    pe = _pe_table(seq, d_model)
    pe2 = jnp.concatenate([pe, pe], axis=0)
    idx = input.reshape(NW, n_chunks, CHUNK).astype(jnp.int32)
    out = _build_sc_call(n_chunks, d_model, seq)(idx, table, pe2)
    return out.reshape(batch, seq, d_model)
